# baseline (speedup 1.0000x reference)
import jax
import jax.numpy as jnp
from jax import lax
from jax.experimental import pallas as pl
from jax.experimental.pallas import tpu as pltpu
from jax.experimental.pallas import tpu_sc as plsc

T = 8192
H = 2048
I = 2048
E = 8
ALPHA = 1.702
BETA = 1.0
LIMIT = 7.0

BT = 512
NTB = T // BT
BM = 256
S_MAX = 2 * T + E * BM
NBLK = S_MAX // BM
EPAD = 128
NC, NS = 2, 16
NW = NC * NS
TPW = T // NW
CH = 16
NCH = TPW // CH


def _router_body(x_ref, gw_ref, gb_ref, e1_ref, e2_ref, r1_ref, r2_ref,
                 w1b_ref, w2b_ref, sb_ref, bex_ref, carry):
    i = pl.program_id(0)

    @pl.when(i == 0)
    def _():
        carry[...] = jnp.zeros((1, EPAD), jnp.float32)

    x = x_ref[...]
    logits = lax.dot_general(x, gw_ref[...], (((1,), (1,)), ((), ())),
                             preferred_element_type=jnp.float32)
    logits = logits + gb_ref[...]
    lane = lax.broadcasted_iota(jnp.int32, (BT, EPAD), 1)
    m1 = jnp.max(logits, axis=1, keepdims=True)
    i1 = jnp.min(jnp.where(logits == m1, lane, EPAD), axis=1, keepdims=True)
    sel1 = lane == i1
    masked = jnp.where(sel1, -3e38, logits)
    m2 = jnp.max(masked, axis=1, keepdims=True)
    i2 = jnp.min(jnp.where(masked == m2, lane, EPAD), axis=1, keepdims=True)
    sel2 = lane == i2
    w1 = jax.nn.sigmoid(m1 - m2)
    oh = (sel1 | sel2).astype(jnp.float32)
    r = lax.broadcasted_iota(jnp.int32, (BT, BT), 0)
    c = lax.broadcasted_iota(jnp.int32, (BT, BT), 1)
    ltri = (r > c).astype(jnp.float32)
    prefix = lax.dot_general(ltri, oh, (((1,), (0,)), ((), ())),
                             preferred_element_type=jnp.float32)
    prefix = prefix + carry[...]
    r1 = jnp.sum(jnp.where(sel1, prefix, 0.0), axis=1, keepdims=True)
    r2 = jnp.sum(jnp.where(sel2, prefix, 0.0), axis=1, keepdims=True)
    new_carry = carry[...] + jnp.sum(oh, axis=0, keepdims=True)
    carry[...] = new_carry
    e1_ref[...] = i1
    e2_ref[...] = i2
    r1_ref[...] = r1.astype(jnp.int32)
    r2_ref[...] = r2.astype(jnp.int32)
    w1b_ref[...] = jnp.broadcast_to(w1, (BT, 16))
    w2b_ref[...] = jnp.broadcast_to(1.0 - w1, (BT, 16))

    @pl.when(i == NTB - 1)
    def _():
        pf = jnp.ceil(new_carry * (1.0 / BM)) * BM
        rr = lax.broadcasted_iota(jnp.int32, (EPAD, EPAD), 0)
        cc = lax.broadcasted_iota(jnp.int32, (EPAD, EPAD), 1)
        sel = jnp.where(cc < rr, jnp.broadcast_to(pf, (EPAD, EPAD)), 0.0)
        ones = jnp.ones((EPAD, EPAD), jnp.float32)
        starts_full = lax.dot_general(sel, ones, (((1,), (0,)), ((), ())),
                                      preferred_element_type=jnp.float32)
        sb_ref[...] = starts_full[:, :16].astype(jnp.int32)
        slot = (cc * BM).astype(jnp.float32)
        ind = jnp.where((starts_full <= slot) & (rr < E), 1.0, 0.0)
        onerow = jnp.ones((1, EPAD), jnp.float32)
        bex = lax.dot_general(onerow, ind, (((1,), (0,)), ((), ())),
                              preferred_element_type=jnp.float32) - 1.0
        bex_ref[...] = jnp.clip(bex, 0.0, float(E - 1)).astype(jnp.int32)


def _router_call(x, gwp, gbp, interpret=False):
    out_shapes = (
        jax.ShapeDtypeStruct((T, 1), jnp.int32),
        jax.ShapeDtypeStruct((T, 1), jnp.int32),
        jax.ShapeDtypeStruct((T, 1), jnp.int32),
        jax.ShapeDtypeStruct((T, 1), jnp.int32),
        jax.ShapeDtypeStruct((T, 16), jnp.float32),
        jax.ShapeDtypeStruct((T, 16), jnp.float32),
        jax.ShapeDtypeStruct((EPAD, 16), jnp.int32),
        jax.ShapeDtypeStruct((1, EPAD), jnp.int32),
    )
    col = pl.BlockSpec((BT, 1), lambda i: (i, 0))
    wcol = pl.BlockSpec((BT, 16), lambda i: (i, 0))
    return pl.pallas_call(
        _router_body,
        grid=(NTB,),
        in_specs=[
            pl.BlockSpec((BT, H), lambda i: (i, 0)),
            pl.BlockSpec((EPAD, H), lambda i: (0, 0)),
            pl.BlockSpec((1, EPAD), lambda i: (0, 0)),
        ],
        out_specs=(col, col, col, col, wcol, wcol,
                   pl.BlockSpec((EPAD, 16), lambda i: (0, 0)),
                   pl.BlockSpec((1, EPAD), lambda i: (0, 0))),
        out_shape=out_shapes,
        scratch_shapes=[pltpu.VMEM((1, EPAD), jnp.float32)],
        interpret=interpret,
    )(x, gwp, gbp)


def _select_start(sbv, ev):
    pos = jnp.zeros((16,), jnp.int32)
    for e in range(1, E):
        pos = pos + jnp.where(ev == e, sbv[e], 0)
    return pos


def _scatter_body(x_hbm, e1_hbm, e2_hbm, r1_hbm, r2_hbm, sb_hbm,
                  xs_hbm,
                  sbv, e1b, e2b, r1b, r2b, bufa, bufb, sga, sgb, ss0, ss1):
    wid = lax.axis_index("s") * NC + lax.axis_index("c")
    base = wid * TPW
    pltpu.sync_copy(sb_hbm.at[pl.ds(0, E)], sbv)
    pltpu.sync_copy(e1_hbm.at[pl.ds(base, TPW)], e1b)
    pltpu.sync_copy(e2_hbm.at[pl.ds(base, TPW)], e2b)
    pltpu.sync_copy(r1_hbm.at[pl.ds(base, TPW)], r1b)
    pltpu.sync_copy(r2_hbm.at[pl.ds(base, TPW)], r2b)
    lanes = lax.iota(jnp.int32, 16)

    for p in range(2):
        def tokidx(c):
            return (base + c * CH + lanes) * 2 + p

        def scat(buf, coff, s0, s1):
            off = pl.ds(coff, CH)
            pos0 = (_select_start(sbv, e1b[off]) + r1b[off]) * 2 + p
            pos1 = (_select_start(sbv, e2b[off]) + r2b[off]) * 2 + p
            d0 = pltpu.async_copy(buf, xs_hbm.at[pos0], s0)
            d1 = pltpu.async_copy(buf, xs_hbm.at[pos1], s1)
            d0.wait()
            d1.wait()

        pltpu.async_copy(x_hbm.at[tokidx(0)], bufa, sga)

        def pair(g, _):
            ca = 2 * g
            cb = ca + 1
            pltpu.make_async_copy(x_hbm.at[pl.ds(0, CH)], bufa, sga).wait()
            dgb = pltpu.async_copy(x_hbm.at[tokidx(cb)], bufb, sgb)
            scat(bufa, ca * CH, ss0, ss1)

            @pl.when(g < NCH // 2 - 1)
            def _():
                pltpu.async_copy(x_hbm.at[tokidx(cb + 1)], bufa, sga)

            dgb.wait()
            scat(bufb, cb * CH, ss0, ss1)
            return 0

        lax.fori_loop(0, NCH // 2, pair, 0)


def _scatter_call(x2, e1, e2, r1, r2, sb):
    mesh = plsc.VectorSubcoreMesh(core_axis_name="c", subcore_axis_name="s",
                                  num_cores=NC, num_subcores=NS)
    k = pl.kernel(
        _scatter_body,
        out_type=jax.ShapeDtypeStruct((2 * S_MAX, H // 2), jnp.float32),
        mesh=mesh,
        scratch_types=[
            pltpu.VMEM((E, 16), jnp.int32),
            pltpu.VMEM((TPW,), jnp.int32),
            pltpu.VMEM((TPW,), jnp.int32),
            pltpu.VMEM((TPW,), jnp.int32),
            pltpu.VMEM((TPW,), jnp.int32),
            pltpu.VMEM((CH, H // 2), jnp.float32),
            pltpu.VMEM((CH, H // 2), jnp.float32),
            pltpu.SemaphoreType.DMA,
            pltpu.SemaphoreType.DMA,
            pltpu.SemaphoreType.DMA,
            pltpu.SemaphoreType.DMA,
        ],
    )
    return k(x2, e1, e2, r1, r2, sb)


def _ffn1_body(be_ref, xs_ref, wg_ref, bg_ref, wl_ref, bl_ref, y_ref):
    x = xs_ref[...].astype(jnp.bfloat16)
    hg = lax.dot_general(x, wg_ref[0], (((1,), (1,)), ((), ())),
                         preferred_element_type=jnp.float32) + bg_ref[0]
    hl = lax.dot_general(x, wl_ref[0], (((1,), (1,)), ((), ())),
                         preferred_element_type=jnp.float32) + bl_ref[0]
    hg = jnp.minimum(hg, LIMIT)
    hl = jnp.clip(hl, -LIMIT, LIMIT)
    y = hg * jax.nn.sigmoid(ALPHA * hg) * (hl + BETA)
    y_ref[...] = y.astype(jnp.bfloat16)


def _ffn1_call(bex, xs, wg, bg, wl, bl, interpret=False):
    I2 = I // 2
    spec = pltpu.PrefetchScalarGridSpec(
        num_scalar_prefetch=1,
        grid=(2, NBLK),
        in_specs=[
            pl.BlockSpec((BM, H), lambda j, b, be: (b, 0)),
            pl.BlockSpec((1, I2, H), lambda j, b, be: (be[b], j, 0)),
            pl.BlockSpec((1, 1, I2), lambda j, b, be: (be[b], 0, j)),
            pl.BlockSpec((1, I2, H), lambda j, b, be: (be[b], j, 0)),
            pl.BlockSpec((1, 1, I2), lambda j, b, be: (be[b], 0, j)),
        ],
        out_specs=pl.BlockSpec((BM, I2), lambda j, b, be: (b, j)),
    )
    return pl.pallas_call(
        _ffn1_body,
        grid_spec=spec,
        out_shape=jax.ShapeDtypeStruct((S_MAX, I), jnp.bfloat16),
        interpret=interpret,
    )(bex, xs, wg, bg, wl, bl)


def _ffn2_body(be_ref, y_ref, w2_ref, b2_ref, o_ref):
    o = lax.dot_general(y_ref[...], w2_ref[0], (((1,), (1,)), ((), ())),
                        preferred_element_type=jnp.float32)
    o_ref[...] = o + b2_ref[0]


def _ffn2_call(bex, y, w2, b2, interpret=False):
    spec = pltpu.PrefetchScalarGridSpec(
        num_scalar_prefetch=1,
        grid=(NBLK,),
        in_specs=[
            pl.BlockSpec((BM, I), lambda b, be: (b, 0)),
            pl.BlockSpec((1, H, I), lambda b, be: (be[b], 0, 0)),
            pl.BlockSpec((1, 1, H), lambda b, be: (be[b], 0, 0)),
        ],
        out_specs=pl.BlockSpec((BM, H), lambda b, be: (b, 0)),
    )
    return pl.pallas_call(
        _ffn2_body,
        grid_spec=spec,
        out_shape=jax.ShapeDtypeStruct((S_MAX, H), jnp.float32),
        interpret=interpret,
    )(bex, y, w2, b2)


def _combine_body(os_hbm, e1_hbm, e2_hbm, r1_hbm, r2_hbm, w1_hbm, w2_hbm,
                  sb_hbm, out_hbm,
                  sbv, e1b, e2b, r1b, r2b, w1a, w2a, w1b, w2b,
                  idxva, idxvb, bufa, bufb, sga, sgb, soa, sob):
    wid = lax.axis_index("s") * NC + lax.axis_index("c")
    base = wid * TPW
    pltpu.sync_copy(sb_hbm.at[pl.ds(0, E)], sbv)
    pltpu.sync_copy(e1_hbm.at[pl.ds(base, TPW)], e1b)
    pltpu.sync_copy(e2_hbm.at[pl.ds(base, TPW)], e2b)
    pltpu.sync_copy(r1_hbm.at[pl.ds(base, TPW)], r1b)
    pltpu.sync_copy(r2_hbm.at[pl.ds(base, TPW)], r2b)
    lanes = lax.iota(jnp.int32, 16)
    H2 = H // 2

    for p in range(2):
        def start_gather(c, idxv, buf, w1x, w2x, sem):
            coff = c * CH
            off = pl.ds(coff, CH)
            pos0 = (_select_start(sbv, e1b[off]) + r1b[off]) * 2 + p
            pos1 = (_select_start(sbv, e2b[off]) + r2b[off]) * 2 + p
            idxv[pl.ds(0, CH)] = pos0
            idxv[pl.ds(CH, CH)] = pos1
            pltpu.async_copy(os_hbm.at[idxv], buf, sem)
            pltpu.sync_copy(w1_hbm.at[pl.ds(base + coff, CH)], w1x)
            pltpu.sync_copy(w2_hbm.at[pl.ds(base + coff, CH)], w2x)

        def finish(c, buf, w1x, w2x, semo):
            coff = c * CH
            wva = [w1x[i] for i in range(CH)]
            wvb = [w2x[i] for i in range(CH)]

            def col(s, _):
                sl = pl.ds(s * 16, 16)
                for i in range(CH):
                    buf[i, sl] = (wva[i] * buf[i, sl]
                                  + wvb[i] * buf[CH + i, sl])
                return 0

            lax.fori_loop(0, H2 // 16, col, 0)
            tokh = (base + coff + lanes) * 2 + p
            return pltpu.async_copy(buf.at[pl.ds(0, CH)], out_hbm.at[tokh],
                                    semo)

        start_gather(0, idxva, bufa, w1a, w2a, sga)

        def pair(g, _):
            ca = 2 * g
            cb = ca + 1
            pltpu.make_async_copy(os_hbm.at[pl.ds(0, 2 * CH)], bufa,
                                  sga).wait()
            start_gather(cb, idxvb, bufb, w1b, w2b, sgb)
            finish(ca, bufa, w1a, w2a, soa).wait()

            @pl.when(g < NCH // 2 - 1)
            def _():
                start_gather(cb + 1, idxva, bufa, w1a, w2a, sga)

            pltpu.make_async_copy(os_hbm.at[pl.ds(0, 2 * CH)], bufb,
                                  sgb).wait()
            finish(cb, bufb, w1b, w2b, sob).wait()
            return 0

        lax.fori_loop(0, NCH // 2, pair, 0)


def _combine_call(os, e1, e2, r1, r2, w1b, w2b, sb):
    mesh = plsc.VectorSubcoreMesh(core_axis_name="c", subcore_axis_name="s",
                                  num_cores=NC, num_subcores=NS)
    k = pl.kernel(
        _combine_body,
        out_type=jax.ShapeDtypeStruct((2 * T, H // 2), jnp.float32),
        mesh=mesh,
        scratch_types=[
            pltpu.VMEM((E, 16), jnp.int32),
            pltpu.VMEM((TPW,), jnp.int32),
            pltpu.VMEM((TPW,), jnp.int32),
            pltpu.VMEM((TPW,), jnp.int32),
            pltpu.VMEM((TPW,), jnp.int32),
            pltpu.VMEM((CH, 16), jnp.float32),
            pltpu.VMEM((CH, 16), jnp.float32),
            pltpu.VMEM((CH, 16), jnp.float32),
            pltpu.VMEM((CH, 16), jnp.float32),
            pltpu.VMEM((2 * CH,), jnp.int32),
            pltpu.VMEM((2 * CH,), jnp.int32),
            pltpu.VMEM((2 * CH, H // 2), jnp.float32),
            pltpu.VMEM((2 * CH, H // 2), jnp.float32),
            pltpu.SemaphoreType.DMA,
            pltpu.SemaphoreType.DMA,
            pltpu.SemaphoreType.DMA,
            pltpu.SemaphoreType.DMA,
        ],
    )
    return k(os, e1, e2, r1, r2, w1b, w2b, sb)


def kernel(hidden_states, gate_w, gate_b, fc1_w, fc1_b, fc2_w, fc2_b):
    x = hidden_states.astype(jnp.float32)
    gwp = jnp.pad(gate_w, ((0, EPAD - E), (0, 0)))
    gbp = jnp.pad(gate_b, (0, EPAD - E), constant_values=-1e30).reshape(1, EPAD)

    e1, e2, r1, r2, w1b, w2b, sb, bex = _router_call(x, gwp, gbp)
    e1f, e2f = e1.reshape(T), e2.reshape(T)
    r1f, r2f = r1.reshape(T), r2.reshape(T)
    bexf = bex.reshape(EPAD)

    xs2 = _scatter_call(x.reshape(2 * T, H // 2), e1f, e2f, r1f, r2f, sb)
    xs = xs2.reshape(S_MAX, H)

    f1 = fc1_w.reshape(E, I, 2, H)
    wg = f1[:, :, 0, :].astype(jnp.bfloat16)
    wl = f1[:, :, 1, :].astype(jnp.bfloat16)
    b1 = fc1_b.reshape(E, I, 2)
    bg = b1[:, :, 0].reshape(E, 1, I)
    bl = b1[:, :, 1].reshape(E, 1, I)

    y = _ffn1_call(bexf, xs, wg, bg, wl, bl)
    os = _ffn2_call(bexf, y, fc2_w.astype(jnp.bfloat16),
                    fc2_b.reshape(E, 1, H))
    os2 = os.reshape(2 * S_MAX, H // 2)
    out2 = _combine_call(os2, e1f, e2f, r1f, r2f, w1b, w2b, sb)
    return out2.reshape(T, H)


# speedup vs baseline: 1.3036x; 1.3036x over previous
import jax
import jax.numpy as jnp
from jax import lax
from jax.experimental import pallas as pl
from jax.experimental.pallas import tpu as pltpu
from jax.experimental.pallas import tpu_sc as plsc

T = 8192
H = 2048
I = 2048
E = 8
ALPHA = 1.702
BETA = 1.0
LIMIT = 7.0

BT = 512
NTB = T // BT
BM = 256
S_MAX = 2 * T + E * BM
NBLK = S_MAX // BM
EPAD = 128
NC, NS = 2, 16
NW = NC * NS
TPW = T // NW
CH = 16
NCH = TPW // CH


def _router_body(x_ref, gw_ref, gb_ref, e1_ref, e2_ref, r1_ref, r2_ref,
                 w1b_ref, w2b_ref, sb_ref, bex_ref, carry):
    i = pl.program_id(0)

    @pl.when(i == 0)
    def _():
        carry[...] = jnp.zeros((1, EPAD), jnp.float32)

    x = x_ref[...]
    logits = lax.dot_general(x, gw_ref[...], (((1,), (1,)), ((), ())),
                             preferred_element_type=jnp.float32)
    logits = logits + gb_ref[...]
    lane = lax.broadcasted_iota(jnp.int32, (BT, EPAD), 1)
    m1 = jnp.max(logits, axis=1, keepdims=True)
    i1 = jnp.min(jnp.where(logits == m1, lane, EPAD), axis=1, keepdims=True)
    sel1 = lane == i1
    masked = jnp.where(sel1, -3e38, logits)
    m2 = jnp.max(masked, axis=1, keepdims=True)
    i2 = jnp.min(jnp.where(masked == m2, lane, EPAD), axis=1, keepdims=True)
    sel2 = lane == i2
    w1 = jax.nn.sigmoid(m1 - m2)
    oh = (sel1 | sel2).astype(jnp.float32)
    r = lax.broadcasted_iota(jnp.int32, (BT, BT), 0)
    c = lax.broadcasted_iota(jnp.int32, (BT, BT), 1)
    ltri = (r > c).astype(jnp.float32)
    prefix = lax.dot_general(ltri, oh, (((1,), (0,)), ((), ())),
                             preferred_element_type=jnp.float32)
    prefix = prefix + carry[...]
    r1 = jnp.sum(jnp.where(sel1, prefix, 0.0), axis=1, keepdims=True)
    r2 = jnp.sum(jnp.where(sel2, prefix, 0.0), axis=1, keepdims=True)
    new_carry = carry[...] + jnp.sum(oh, axis=0, keepdims=True)
    carry[...] = new_carry
    e1_ref[...] = i1
    e2_ref[...] = i2
    r1_ref[...] = r1.astype(jnp.int32)
    r2_ref[...] = r2.astype(jnp.int32)
    w1b_ref[...] = jnp.broadcast_to(w1, (BT, 16))
    w2b_ref[...] = jnp.broadcast_to(1.0 - w1, (BT, 16))

    @pl.when(i == NTB - 1)
    def _():
        pf = jnp.ceil(new_carry * (1.0 / BM)) * BM
        rr = lax.broadcasted_iota(jnp.int32, (EPAD, EPAD), 0)
        cc = lax.broadcasted_iota(jnp.int32, (EPAD, EPAD), 1)
        sel = jnp.where(cc < rr, jnp.broadcast_to(pf, (EPAD, EPAD)), 0.0)
        ones = jnp.ones((EPAD, EPAD), jnp.float32)
        starts_full = lax.dot_general(sel, ones, (((1,), (0,)), ((), ())),
                                      preferred_element_type=jnp.float32)
        sb_ref[...] = starts_full[:, :16].astype(jnp.int32)
        slot = (cc * BM).astype(jnp.float32)
        ind = jnp.where((starts_full <= slot) & (rr < E), 1.0, 0.0)
        onerow = jnp.ones((1, EPAD), jnp.float32)
        bex = lax.dot_general(onerow, ind, (((1,), (0,)), ((), ())),
                              preferred_element_type=jnp.float32) - 1.0
        bex_ref[...] = jnp.clip(bex, 0.0, float(E - 1)).astype(jnp.int32)


def _router_call(x, gwp, gbp, interpret=False):
    out_shapes = (
        jax.ShapeDtypeStruct((T, 1), jnp.int32),
        jax.ShapeDtypeStruct((T, 1), jnp.int32),
        jax.ShapeDtypeStruct((T, 1), jnp.int32),
        jax.ShapeDtypeStruct((T, 1), jnp.int32),
        jax.ShapeDtypeStruct((T, 16), jnp.float32),
        jax.ShapeDtypeStruct((T, 16), jnp.float32),
        jax.ShapeDtypeStruct((EPAD, 16), jnp.int32),
        jax.ShapeDtypeStruct((1, EPAD), jnp.int32),
    )
    col = pl.BlockSpec((BT, 1), lambda i: (i, 0))
    wcol = pl.BlockSpec((BT, 16), lambda i: (i, 0))
    return pl.pallas_call(
        _router_body,
        grid=(NTB,),
        in_specs=[
            pl.BlockSpec((BT, H), lambda i: (i, 0)),
            pl.BlockSpec((EPAD, H), lambda i: (0, 0)),
            pl.BlockSpec((1, EPAD), lambda i: (0, 0)),
        ],
        out_specs=(col, col, col, col, wcol, wcol,
                   pl.BlockSpec((EPAD, 16), lambda i: (0, 0)),
                   pl.BlockSpec((1, EPAD), lambda i: (0, 0))),
        out_shape=out_shapes,
        scratch_shapes=[pltpu.VMEM((1, EPAD), jnp.float32)],
        interpret=interpret,
    )(x, gwp, gbp)


def _select_start(sbv, ev):
    pos = jnp.zeros((16,), jnp.int32)
    for e in range(1, E):
        pos = pos + jnp.where(ev == e, sbv[e], 0)
    return pos


def _scatter_body(x_hbm, e1_hbm, e2_hbm, r1_hbm, r2_hbm, sb_hbm,
                  xs_hbm,
                  sbv, e1b, e2b, r1b, r2b, bufx, sga, ss0, ss1):
    wid = lax.axis_index("s") * NC + lax.axis_index("c")
    base = wid * TPW
    pltpu.sync_copy(sb_hbm.at[pl.ds(0, E)], sbv)
    pltpu.sync_copy(e1_hbm.at[pl.ds(base, TPW)], e1b)
    pltpu.sync_copy(e2_hbm.at[pl.ds(base, TPW)], e2b)
    pltpu.sync_copy(r1_hbm.at[pl.ds(base, TPW)], r1b)
    pltpu.sync_copy(r2_hbm.at[pl.ds(base, TPW)], r2b)

    def chunk(ch, _):
        coff = ch * CH
        off = pl.ds(coff, CH)
        pltpu.async_copy(x_hbm.at[pl.ds(base + coff, CH)], bufx, sga).wait()
        pos0 = _select_start(sbv, e1b[off]) + r1b[off]
        pos1 = _select_start(sbv, e2b[off]) + r2b[off]
        d0 = pltpu.async_copy(bufx, xs_hbm.at[pos0], ss0)
        d1 = pltpu.async_copy(bufx, xs_hbm.at[pos1], ss1)
        d0.wait()
        d1.wait()
        return 0

    lax.fori_loop(0, NCH, chunk, 0)


def _scatter_call(x, e1, e2, r1, r2, sb):
    mesh = plsc.VectorSubcoreMesh(core_axis_name="c", subcore_axis_name="s",
                                  num_cores=NC, num_subcores=NS)
    k = pl.kernel(
        _scatter_body,
        out_type=jax.ShapeDtypeStruct((S_MAX, H), jnp.float32),
        mesh=mesh,
        scratch_types=[
            pltpu.VMEM((E, 16), jnp.int32),
            pltpu.VMEM((TPW,), jnp.int32),
            pltpu.VMEM((TPW,), jnp.int32),
            pltpu.VMEM((TPW,), jnp.int32),
            pltpu.VMEM((TPW,), jnp.int32),
            pltpu.VMEM((CH, H), jnp.float32),
            pltpu.SemaphoreType.DMA,
            pltpu.SemaphoreType.DMA,
            pltpu.SemaphoreType.DMA,
        ],
    )
    return k(x, e1, e2, r1, r2, sb)


def _ffn1_body(be_ref, xs_ref, wg_ref, bg_ref, wl_ref, bl_ref, y_ref):
    x = xs_ref[...].astype(jnp.bfloat16)
    hg = lax.dot_general(x, wg_ref[0], (((1,), (1,)), ((), ())),
                         preferred_element_type=jnp.float32) + bg_ref[0]
    hl = lax.dot_general(x, wl_ref[0], (((1,), (1,)), ((), ())),
                         preferred_element_type=jnp.float32) + bl_ref[0]
    hg = jnp.minimum(hg, LIMIT)
    hl = jnp.clip(hl, -LIMIT, LIMIT)
    y = hg * jax.nn.sigmoid(ALPHA * hg) * (hl + BETA)
    y_ref[...] = y.astype(jnp.bfloat16)


def _ffn1_call(bex, xs, wg, bg, wl, bl, interpret=False):
    I2 = I // 2
    spec = pltpu.PrefetchScalarGridSpec(
        num_scalar_prefetch=1,
        grid=(2, NBLK),
        in_specs=[
            pl.BlockSpec((BM, H), lambda j, b, be: (b, 0)),
            pl.BlockSpec((1, I2, H), lambda j, b, be: (be[b], j, 0)),
            pl.BlockSpec((1, 1, I2), lambda j, b, be: (be[b], 0, j)),
            pl.BlockSpec((1, I2, H), lambda j, b, be: (be[b], j, 0)),
            pl.BlockSpec((1, 1, I2), lambda j, b, be: (be[b], 0, j)),
        ],
        out_specs=pl.BlockSpec((BM, I2), lambda j, b, be: (b, j)),
    )
    return pl.pallas_call(
        _ffn1_body,
        grid_spec=spec,
        out_shape=jax.ShapeDtypeStruct((S_MAX, I), jnp.bfloat16),
        interpret=interpret,
    )(bex, xs, wg, bg, wl, bl)


def _ffn2_body(be_ref, y_ref, w2_ref, b2_ref, o_ref):
    o = lax.dot_general(y_ref[...], w2_ref[0], (((1,), (1,)), ((), ())),
                        preferred_element_type=jnp.float32)
    o_ref[...] = o + b2_ref[0]


def _ffn2_call(bex, y, w2, b2, interpret=False):
    spec = pltpu.PrefetchScalarGridSpec(
        num_scalar_prefetch=1,
        grid=(NBLK,),
        in_specs=[
            pl.BlockSpec((BM, I), lambda b, be: (b, 0)),
            pl.BlockSpec((1, H, I), lambda b, be: (be[b], 0, 0)),
            pl.BlockSpec((1, 1, H), lambda b, be: (be[b], 0, 0)),
        ],
        out_specs=pl.BlockSpec((BM, H), lambda b, be: (b, 0)),
    )
    return pl.pallas_call(
        _ffn2_body,
        grid_spec=spec,
        out_shape=jax.ShapeDtypeStruct((S_MAX, H), jnp.float32),
        interpret=interpret,
    )(bex, y, w2, b2)


def _combine_body(os_hbm, e1_hbm, e2_hbm, r1_hbm, r2_hbm, w1_hbm, w2_hbm,
                  sb_hbm, out_hbm,
                  sbv, e1b, e2b, r1b, r2b, w1bv, w2bv, bufa, bufb, sem):
    wid = lax.axis_index("s") * NC + lax.axis_index("c")
    base = wid * TPW
    pltpu.sync_copy(sb_hbm.at[pl.ds(0, E)], sbv)
    pltpu.sync_copy(e1_hbm.at[pl.ds(base, TPW)], e1b)
    pltpu.sync_copy(e2_hbm.at[pl.ds(base, TPW)], e2b)
    pltpu.sync_copy(r1_hbm.at[pl.ds(base, TPW)], r1b)
    pltpu.sync_copy(r2_hbm.at[pl.ds(base, TPW)], r2b)

    def chunk(ch, _):
        coff = ch * CH
        off = pl.ds(coff, CH)
        pltpu.sync_copy(w1_hbm.at[pl.ds(base + coff, CH)], w1bv)
        pltpu.sync_copy(w2_hbm.at[pl.ds(base + coff, CH)], w2bv)
        pos0 = _select_start(sbv, e1b[off]) + r1b[off]
        pos1 = _select_start(sbv, e2b[off]) + r2b[off]
        da = pltpu.async_copy(os_hbm.at[pos0], bufa, sem)
        db = pltpu.async_copy(os_hbm.at[pos1], bufb, sem)
        da.wait()
        db.wait()
        wva = [w1bv[i] for i in range(CH)]
        wvb = [w2bv[i] for i in range(CH)]

        def col(s, _):
            sl = pl.ds(s * 16, 16)
            for i in range(CH):
                bufa[i, sl] = wva[i] * bufa[i, sl] + wvb[i] * bufb[i, sl]
            return 0

        lax.fori_loop(0, H // 16, col, 0)
        pltpu.sync_copy(bufa, out_hbm.at[pl.ds(base + coff, CH)])
        return 0

    lax.fori_loop(0, NCH, chunk, 0)


def _combine_call(os, e1, e2, r1, r2, w1b, w2b, sb):
    mesh = plsc.VectorSubcoreMesh(core_axis_name="c", subcore_axis_name="s",
                                  num_cores=NC, num_subcores=NS)
    k = pl.kernel(
        _combine_body,
        out_type=jax.ShapeDtypeStruct((T, H), jnp.float32),
        mesh=mesh,
        scratch_types=[
            pltpu.VMEM((E, 16), jnp.int32),
            pltpu.VMEM((TPW,), jnp.int32),
            pltpu.VMEM((TPW,), jnp.int32),
            pltpu.VMEM((TPW,), jnp.int32),
            pltpu.VMEM((TPW,), jnp.int32),
            pltpu.VMEM((CH, 16), jnp.float32),
            pltpu.VMEM((CH, 16), jnp.float32),
            pltpu.VMEM((CH, H), jnp.float32),
            pltpu.VMEM((CH, H), jnp.float32),
            pltpu.SemaphoreType.DMA,
        ],
    )
    return k(os, e1, e2, r1, r2, w1b, w2b, sb)


def kernel(hidden_states, gate_w, gate_b, fc1_w, fc1_b, fc2_w, fc2_b):
    x = hidden_states.astype(jnp.float32)
    gwp = jnp.pad(gate_w, ((0, EPAD - E), (0, 0)))
    gbp = jnp.pad(gate_b, (0, EPAD - E), constant_values=-1e30).reshape(1, EPAD)

    e1, e2, r1, r2, w1b, w2b, sb, bex = _router_call(x, gwp, gbp)
    e1f, e2f = e1.reshape(T), e2.reshape(T)
    r1f, r2f = r1.reshape(T), r2.reshape(T)
    bexf = bex.reshape(EPAD)

    xs = _scatter_call(x, e1f, e2f, r1f, r2f, sb)

    f1 = fc1_w.reshape(E, I, 2, H)
    wg = f1[:, :, 0, :].astype(jnp.bfloat16)
    wl = f1[:, :, 1, :].astype(jnp.bfloat16)
    b1 = fc1_b.reshape(E, I, 2)
    bg = b1[:, :, 0].reshape(E, 1, I)
    bl = b1[:, :, 1].reshape(E, 1, I)

    y = _ffn1_call(bexf, xs, wg, bg, wl, bl)
    os = _ffn2_call(bexf, y, fc2_w.astype(jnp.bfloat16),
                    fc2_b.reshape(E, 1, H))
    return _combine_call(os, e1f, e2f, r1f, r2f, w1b, w2b, sb)
